# baseline (speedup 1.0000x reference)
import functools

import numpy as np
import jax
import jax.numpy as jnp
from jax import lax
from jax.experimental import pallas as pl
from jax.experimental.pallas import tpu as pltpu
from jax.experimental.pallas import tpu_sc as plsc

F32 = jnp.float32
BF16 = jnp.bfloat16


def _mm(x, w):
    return jnp.dot(x.astype(BF16), w, preferred_element_type=F32)


def _sc_gather(table, idx):
    R, D = table.shape
    (Bt,) = idx.shape
    W = min(32, Bt // 8)
    bpw = Bt // W
    c = bpw
    while c * D * 4 > 196608 or c > 128:
        c //= 2
    nch = bpw // c
    mesh = plsc.VectorSubcoreMesh(core_axis_name="c", subcore_axis_name="s")

    @functools.partial(
        pl.kernel,
        mesh=mesh,
        out_type=jax.ShapeDtypeStruct((Bt, D), F32),
        scratch_types=[
            pltpu.VMEM((2, c), jnp.int32),
            pltpu.VMEM((2, c, D), F32),
            pltpu.SemaphoreType.DMA,
            pltpu.SemaphoreType.DMA,
        ],
    )
    def gk(table_hbm, idx_hbm, out_hbm, idx_v, rows_v, sem0, sem1):
        wid = lax.axis_index("s") * 2 + lax.axis_index("c")
        sems = (sem0, sem1)

        @pl.when(wid < W)
        def _():
            base0 = wid * bpw
            pltpu.sync_copy(idx_hbm.at[pl.ds(base0, c)], idx_v.at[0])
            cp0 = pltpu.async_copy(table_hbm.at[idx_v.at[0]], rows_v.at[0], sem0)

            def pair(t, carry):
                i0 = 2 * t
                pltpu.make_async_copy(table_hbm.at[idx_v.at[0]], rows_v.at[0],
                                      sem0).wait()

                @pl.when(i0 + 1 < nch)
                def _():
                    pltpu.sync_copy(
                        idx_hbm.at[pl.ds(base0 + (i0 + 1) * c, c)], idx_v.at[1])
                    pltpu.async_copy(table_hbm.at[idx_v.at[1]], rows_v.at[1], sem1)

                pltpu.sync_copy(rows_v.at[0], out_hbm.at[pl.ds(base0 + i0 * c, c)])

                @pl.when(i0 + 1 < nch)
                def _():
                    pltpu.make_async_copy(table_hbm.at[idx_v.at[1]], rows_v.at[1],
                                          sem1).wait()

                    @pl.when(i0 + 2 < nch)
                    def _():
                        pltpu.sync_copy(
                            idx_hbm.at[pl.ds(base0 + (i0 + 2) * c, c)], idx_v.at[0])
                        pltpu.async_copy(table_hbm.at[idx_v.at[0]], rows_v.at[0],
                                         sem0)

                    pltpu.sync_copy(rows_v.at[1],
                                    out_hbm.at[pl.ds(base0 + (i0 + 1) * c, c)])
                return carry

            lax.fori_loop(0, (nch + 1) // 2, pair, 0)

    return gk(table, idx)


def _knn(q_xyz, base_xyz, k):
    B, M, _ = q_xyz.shape
    N = base_xyz.shape[1]
    BM = min(M, 256)
    base_t = jnp.transpose(base_xyz, (0, 2, 1))

    def body(q_ref, bt_ref, o_ref):
        b = pl.program_id(0)
        q = q_ref[0]
        bt = bt_ref[0]
        q0, q1, q2 = q[:, 0:1], q[:, 1:2], q[:, 2:3]
        b0, b1, b2 = bt[0:1, :], bt[1:2, :], bt[2:3, :]
        sq = (q0 * q0 + q1 * q1) + q2 * q2
        sb = (b0 * b0 + b1 * b1) + b2 * b2
        r = lambda t: t.astype(BF16).astype(F32)
        dot = (r(q0) * r(b0) + r(q1) * r(b1)) + r(q2) * r(b2)
        d = (sq + sb) - 2.0 * dot
        iota = lax.broadcasted_iota(jnp.int32, (BM, N), 1)
        kio = lax.broadcasted_iota(jnp.int32, (BM, k), 1)
        acc = jnp.zeros((BM, k), jnp.int32)
        for j in range(k):
            m = jnp.min(d, axis=1, keepdims=True)
            am = jnp.min(jnp.where(d == m, iota, N), axis=1, keepdims=True)
            acc = jnp.where(kio == j, am, acc)
            d = jnp.where(iota == am, jnp.inf, d)
        o_ref[0] = acc + b * N

    return pl.pallas_call(
        body,
        grid=(B, M // BM),
        in_specs=[
            pl.BlockSpec((1, BM, 3), lambda b, i: (b, i, 0)),
            pl.BlockSpec((1, 3, N), lambda b, i: (b, 0, 0)),
        ],
        out_specs=pl.BlockSpec((1, BM, k), lambda b, i: (b, i, 0)),
        out_shape=jax.ShapeDtypeStruct((B, M, k), jnp.int32),
    )(q_xyz, base_t)


def _fps(xyz, npoint):
    B, N, _ = xyz.shape
    LN = min(N, 128)
    NS = N // LN
    xyz_r = jnp.transpose(xyz, (0, 2, 1)).reshape(B, 3, NS, LN)

    def body(xr_ref, o_ref):
        iota2 = (lax.broadcasted_iota(jnp.int32, (NS, LN), 0) * LN
                 + lax.broadcasted_iota(jnp.int32, (NS, LN), 1))
        iota_p = lax.broadcasted_iota(jnp.int32, (1, npoint), 1)

        def step(i, carry):
            fars, dists, cents = carry
            new = []
            for b in range(B):
                far, dist, cent = fars[b], dists[b], cents[b]
                cent = jnp.where(iota_p == i, far, cent)
                sel = iota2 == far
                cx = jnp.max(jnp.where(sel, xr_ref[b, 0], -jnp.inf))
                cy = jnp.max(jnp.where(sel, xr_ref[b, 1], -jnp.inf))
                cz = jnp.max(jnp.where(sel, xr_ref[b, 2], -jnp.inf))
                dx = xr_ref[b, 0] - cx
                dy = xr_ref[b, 1] - cy
                dz = xr_ref[b, 2] - cz
                dd = (dx * dx + dy * dy) + dz * dz
                dist = jnp.minimum(dist, dd)
                m = jnp.max(dist)
                far2 = jnp.min(jnp.where(dist == m, iota2, N)).astype(jnp.int32)
                new.append((far2, dist, cent))
            return (tuple(t[0] for t in new), tuple(t[1] for t in new),
                    tuple(t[2] for t in new))

        init = (tuple(jnp.int32(0) for _ in range(B)),
                tuple(jnp.full((NS, LN), 1e10, F32) for _ in range(B)),
                tuple(jnp.zeros((1, npoint), jnp.int32) for _ in range(B)))
        _, _, cents = lax.fori_loop(0, npoint, step, init)
        for b in range(B):
            o_ref[b] = cents[b] + b * N

    return pl.pallas_call(
        body,
        grid=(1,),
        in_specs=[
            pl.BlockSpec((B, 3, NS, LN), lambda i: (0, 0, 0, 0)),
        ],
        out_specs=pl.BlockSpec((B, 1, npoint), lambda i: (0, 0, 0)),
        out_shape=jax.ShapeDtypeStruct((B, 1, npoint), jnp.int32),
    )(xyz_r)


def _embed(x, p1, p2):
    B, N, Ci = x.shape
    C1 = p1['w'].shape[0]
    C2 = p2['w'].shape[0]

    def body(x_ref, w1_ref, b1_ref, w2_ref, b2_ref, o_ref):
        h = jnp.maximum(_mm(x_ref[0], w1_ref[...]) + b1_ref[...], 0.0)
        o_ref[0] = _mm(h, w2_ref[...]) + b2_ref[...]

    return pl.pallas_call(
        body,
        grid=(B,),
        in_specs=[
            pl.BlockSpec((1, N, Ci), lambda b: (b, 0, 0)),
            pl.BlockSpec((Ci, C1), lambda b: (0, 0)),
            pl.BlockSpec((1, C1), lambda b: (0, 0)),
            pl.BlockSpec((C1, C2), lambda b: (0, 0)),
            pl.BlockSpec((1, C2), lambda b: (0, 0)),
        ],
        out_specs=pl.BlockSpec((1, N, C2), lambda b: (b, 0, 0)),
        out_shape=jax.ShapeDtypeStruct((B, N, C2), F32),
    )(x, p1['w'].T.astype(BF16), p1['b'].reshape(1, -1),
      p2['w'].T.astype(BF16), p2['b'].reshape(1, -1))


_DM = 512
_TABW = 1152


def _pt_pre(p, feats, xyz):
    B, N, C = feats.shape
    BM = min(N, 256)

    def body(f_ref, xyz_ref, w1_ref, b1_ref, wphi_ref, wpsi_ref, wal_ref,
             q_ref, tab_ref):
        xi = _mm(f_ref[0], w1_ref[...]) + b1_ref[...]
        q_ref[0] = _mm(xi, wphi_ref[...])
        tab_ref[:, 0:_DM] = _mm(xi, wpsi_ref[...])
        tab_ref[:, _DM:2 * _DM] = _mm(xi, wal_ref[...])
        xyzp = jnp.concatenate(
            [xyz_ref[0], jnp.zeros((BM, 125), F32)], axis=1)
        tab_ref[:, 2 * _DM:_TABW] = xyzp

    q, tab = pl.pallas_call(
        body,
        grid=(B, N // BM),
        in_specs=[
            pl.BlockSpec((1, BM, C), lambda b, i: (b, i, 0)),
            pl.BlockSpec((1, BM, 3), lambda b, i: (b, i, 0)),
            pl.BlockSpec((C, _DM), lambda b, i: (0, 0)),
            pl.BlockSpec((1, _DM), lambda b, i: (0, 0)),
            pl.BlockSpec((_DM, _DM), lambda b, i: (0, 0)),
            pl.BlockSpec((_DM, _DM), lambda b, i: (0, 0)),
            pl.BlockSpec((_DM, _DM), lambda b, i: (0, 0)),
        ],
        out_specs=[
            pl.BlockSpec((1, BM, _DM), lambda b, i: (b, i, 0)),
            pl.BlockSpec((BM, _TABW), lambda b, i: (b * (N // BM) + i, 0)),
        ],
        out_shape=[
            jax.ShapeDtypeStruct((B, N, _DM), F32),
            jax.ShapeDtypeStruct((B * N, _TABW), F32),
        ],
    )(feats, xyz,
      p['fc1']['w'].T.astype(BF16), p['fc1']['b'].reshape(1, -1),
      p['phi']['w'].T.astype(BF16), p['psi']['w'].T.astype(BF16),
      p['alpha']['w'].T.astype(BF16))
    return q, tab


def _pt_attn(p, q, xyz, feats, g, k):
    B, N, C = feats.shape
    BM = min(N, 64)
    sqrt_d = float(np.sqrt(_DM))

    def body(q_ref, xyz_ref, f_ref, g_ref, wd1_ref, bd1_ref, wd2_ref, bd2_ref,
             wg1_ref, bg1_ref, wg2_ref, bg2_ref, wf2_ref, bf2_ref, o_ref):
        g2 = g_ref[...]
        kk = g2[:, 0:_DM]
        v = g2[:, _DM:2 * _DM]
        xj = g2[:, 2 * _DM:2 * _DM + 3]
        rel = (xyz_ref[0].reshape(BM, 1, 3) - xj.reshape(BM, k, 3)
               ).reshape(BM * k, 3)
        p1 = jnp.maximum(_mm(rel, wd1_ref[...]) + bd1_ref[...], 0.0)
        pos = _mm(p1, wd2_ref[...]) + bd2_ref[...]
        u = (q_ref[0].reshape(BM, 1, _DM) - kk.reshape(BM, k, _DM)
             ) + pos.reshape(BM, k, _DM)
        a1 = jnp.maximum(
            _mm(u.reshape(BM * k, _DM), wg1_ref[...]) + bg1_ref[...], 0.0)
        e = _mm(a1, wg2_ref[...]) + bg2_ref[...]
        e3 = (e / sqrt_d).reshape(BM, k, _DM)
        m = jnp.max(e3, axis=1, keepdims=True)
        ex = jnp.exp(e3 - m)
        s = jnp.sum(ex, axis=1, keepdims=True)
        attn = ex / s
        res = jnp.sum(attn * (v.reshape(BM, k, _DM) + pos.reshape(BM, k, _DM)),
                      axis=1)
        o_ref[0] = _mm(res, wf2_ref[...]) + bf2_ref[...] + f_ref[0]

    return pl.pallas_call(
        body,
        grid=(B, N // BM),
        in_specs=[
            pl.BlockSpec((1, BM, _DM), lambda b, i: (b, i, 0)),
            pl.BlockSpec((1, BM, 3), lambda b, i: (b, i, 0)),
            pl.BlockSpec((1, BM, C), lambda b, i: (b, i, 0)),
            pl.BlockSpec((BM * k, _TABW), lambda b, i: (b * (N // BM) + i, 0)),
            pl.BlockSpec((3, _DM), lambda b, i: (0, 0)),
            pl.BlockSpec((1, _DM), lambda b, i: (0, 0)),
            pl.BlockSpec((_DM, _DM), lambda b, i: (0, 0)),
            pl.BlockSpec((1, _DM), lambda b, i: (0, 0)),
            pl.BlockSpec((_DM, _DM), lambda b, i: (0, 0)),
            pl.BlockSpec((1, _DM), lambda b, i: (0, 0)),
            pl.BlockSpec((_DM, _DM), lambda b, i: (0, 0)),
            pl.BlockSpec((1, _DM), lambda b, i: (0, 0)),
            pl.BlockSpec((_DM, C), lambda b, i: (0, 0)),
            pl.BlockSpec((1, C), lambda b, i: (0, 0)),
        ],
        out_specs=pl.BlockSpec((1, BM, C), lambda b, i: (b, i, 0)),
        out_shape=jax.ShapeDtypeStruct((B, N, C), F32),
    )(q, xyz, feats, g,
      p['delta1']['w'].T.astype(BF16), p['delta1']['b'].reshape(1, -1),
      p['delta2']['w'].T.astype(BF16), p['delta2']['b'].reshape(1, -1),
      p['gamma1']['w'].T.astype(BF16), p['gamma1']['b'].reshape(1, -1),
      p['gamma2']['w'].T.astype(BF16), p['gamma2']['b'].reshape(1, -1),
      p['fc2']['w'].T.astype(BF16), p['fc2']['b'].reshape(1, -1))


def _pt_block(p, xyz, feats):
    B, N, C = feats.shape
    k = min(16, N)
    idx = _knn(xyz, xyz, k)
    q, tab = _pt_pre(p, feats, xyz)
    g = _sc_gather(tab, idx.reshape(-1))
    return _pt_attn(p, q, xyz, feats, g, k)


def _td_mlp(p, g1, new_xyz2, B, npoint, ns, C, gw):
    l1, l2 = p['layers']
    ch = l1['w'].shape[0]
    chp = max(ch, 128)
    pad = ((0, 0), (0, chp - ch))
    R = B * npoint
    R16 = R * ns
    w1x = jnp.pad(l1['w'][:, :3].T, pad)
    w1p = jnp.pad(l1['w'][:, 3:].T, pad)

    def body(g_ref, nx_ref, wx_ref, wp_ref, b1_ref, ga1_ref, be1_ref,
             w2_ref, b2_ref, ga2_ref, be2_ref, o_ref):
        g2 = g_ref[...]
        pts = g2[:, 0:C]
        xyz_j = g2[:, C:C + 3]
        rel = (xyz_j.reshape(R, ns, 3) - nx_ref[...].reshape(R, 1, 3)
               ).reshape(R16, 3)
        xpre = (_mm(pts, wp_ref[...]) + _mm(rel, wx_ref[...])) + b1_ref[...]
        m1 = jnp.mean(xpre, axis=0, keepdims=True)
        v1 = jnp.mean((xpre - m1) ** 2, axis=0, keepdims=True)
        h = jnp.maximum(
            ga1_ref[...] * (xpre - m1) / jnp.sqrt(v1 + 1e-5) + be1_ref[...], 0.0)
        y = _mm(h, w2_ref[...]) + b2_ref[...]
        m2 = jnp.mean(y, axis=0, keepdims=True)
        v2 = jnp.mean((y - m2) ** 2, axis=0, keepdims=True)
        z = jnp.maximum(
            ga2_ref[...] * (y - m2) / jnp.sqrt(v2 + 1e-5) + be2_ref[...], 0.0)
        o_ref[...] = jnp.max(z.reshape(R, ns, chp), axis=1)

    return pl.pallas_call(
        body,
        grid=(1,),
        in_specs=[
            pl.BlockSpec((R16, gw), lambda i: (0, 0)),
            pl.BlockSpec((R, 3), lambda i: (0, 0)),
            pl.BlockSpec((3, chp), lambda i: (0, 0)),
            pl.BlockSpec((C, chp), lambda i: (0, 0)),
            pl.BlockSpec((1, chp), lambda i: (0, 0)),
            pl.BlockSpec((1, chp), lambda i: (0, 0)),
            pl.BlockSpec((1, chp), lambda i: (0, 0)),
            pl.BlockSpec((chp, chp), lambda i: (0, 0)),
            pl.BlockSpec((1, chp), lambda i: (0, 0)),
            pl.BlockSpec((1, chp), lambda i: (0, 0)),
            pl.BlockSpec((1, chp), lambda i: (0, 0)),
        ],
        out_specs=pl.BlockSpec((R, chp), lambda i: (0, 0)),
        out_shape=jax.ShapeDtypeStruct((R, chp), F32),
    )(g1, new_xyz2,
      w1x.astype(BF16), w1p.astype(BF16),
      jnp.pad(l1['b'].reshape(1, -1), pad),
      jnp.pad(l1['gamma'].reshape(1, -1), pad),
      jnp.pad(l1['beta'].reshape(1, -1), pad),
      jnp.pad(l2['w'].T, ((0, chp - ch), (0, chp - ch))).astype(BF16),
      jnp.pad(l2['b'].reshape(1, -1), pad),
      jnp.pad(l2['gamma'].reshape(1, -1), pad),
      jnp.pad(l2['beta'].reshape(1, -1), pad))


def kernel(input_point_cloud, params):
    x = input_point_cloud
    B, N, _ = x.shape
    xyz = x[..., :3]
    h = _embed(x, params['fc1a'], params['fc1b'])

    points = None
    feats = []
    for i in range(5):
        feat_in = h if i == 0 else points
        p = params['t1'] if i == 0 else params['trs'][i - 1]
        C = feat_in.shape[2]
        Np = feat_in.shape[1]
        k = min(16, Np)
        idx = _knn(xyz, xyz, k)
        q, tab = _pt_pre(p, feat_in, xyz)
        g = _sc_gather(tab, idx.reshape(-1))

        if i < 4:
            npoint = N // 4 ** (i + 1)
            fidx = _fps(xyz, npoint)
            xyz_pad = jnp.pad(xyz.reshape(B * Np, 3), ((0, 0), (0, 125)))
            nxyz_p = _sc_gather(xyz_pad, fidx.reshape(-1))
            new_xyz = nxyz_p[:, :3].reshape(B, npoint, 3)
            ns = min(16, Np)
            idx_td = _knn(new_xyz, xyz, ns)

        points = _pt_attn(p, q, xyz, feat_in, g, k)
        feats.append((xyz, points))

        if i < 4:
            td = params['tds'][i]
            ch = td['layers'][0]['w'].shape[0]
            gw = (C + 3 + 127) // 128 * 128
            tab_td = jnp.pad(
                jnp.concatenate(
                    [points.reshape(B * Np, C), xyz.reshape(B * Np, 3)],
                    axis=1),
                ((0, 0), (0, gw - C - 3)))
            g1 = _sc_gather(tab_td, idx_td.reshape(-1))
            newp = _td_mlp(td, g1, new_xyz.reshape(B * npoint, 3),
                           B, npoint, ns, C, gw)
            xyz = new_xyz
            points = newp[:, :ch].reshape(B, npoint, ch)

    return points, feats


# speedup vs baseline: 1.1980x; 1.1980x over previous
import functools

import numpy as np
import jax
import jax.numpy as jnp
from jax import lax
from jax.experimental import pallas as pl
from jax.experimental.pallas import tpu as pltpu
from jax.experimental.pallas import tpu_sc as plsc

F32 = jnp.float32
BF16 = jnp.bfloat16


def _mm(x, w):
    return jnp.dot(x.astype(BF16), w, preferred_element_type=F32)


def _sc_gather(table, idx):
    R, D = table.shape
    (Bt,) = idx.shape
    W = min(32, Bt // 8)
    bpw = Bt // W
    c = bpw
    while c * D * 4 > 196608 or c > 128:
        c //= 2
    nch = bpw // c
    mesh = plsc.VectorSubcoreMesh(core_axis_name="c", subcore_axis_name="s")

    @functools.partial(
        pl.kernel,
        mesh=mesh,
        out_type=jax.ShapeDtypeStruct((Bt, D), F32),
        scratch_types=[
            pltpu.VMEM((2, c), jnp.int32),
            pltpu.VMEM((2, c, D), F32),
            pltpu.SemaphoreType.DMA,
            pltpu.SemaphoreType.DMA,
        ],
    )
    def gk(table_hbm, idx_hbm, out_hbm, idx_v, rows_v, sem0, sem1):
        wid = lax.axis_index("s") * 2 + lax.axis_index("c")
        sems = (sem0, sem1)

        @pl.when(wid < W)
        def _():
            base0 = wid * bpw
            pltpu.sync_copy(idx_hbm.at[pl.ds(base0, c)], idx_v.at[0])
            cp0 = pltpu.async_copy(table_hbm.at[idx_v.at[0]], rows_v.at[0], sem0)

            def pair(t, carry):
                i0 = 2 * t
                pltpu.make_async_copy(table_hbm.at[idx_v.at[0]], rows_v.at[0],
                                      sem0).wait()

                @pl.when(i0 + 1 < nch)
                def _():
                    pltpu.sync_copy(
                        idx_hbm.at[pl.ds(base0 + (i0 + 1) * c, c)], idx_v.at[1])
                    pltpu.async_copy(table_hbm.at[idx_v.at[1]], rows_v.at[1], sem1)

                pltpu.sync_copy(rows_v.at[0], out_hbm.at[pl.ds(base0 + i0 * c, c)])

                @pl.when(i0 + 1 < nch)
                def _():
                    pltpu.make_async_copy(table_hbm.at[idx_v.at[1]], rows_v.at[1],
                                          sem1).wait()

                    @pl.when(i0 + 2 < nch)
                    def _():
                        pltpu.sync_copy(
                            idx_hbm.at[pl.ds(base0 + (i0 + 2) * c, c)], idx_v.at[0])
                        pltpu.async_copy(table_hbm.at[idx_v.at[0]], rows_v.at[0],
                                         sem0)

                    pltpu.sync_copy(rows_v.at[1],
                                    out_hbm.at[pl.ds(base0 + (i0 + 1) * c, c)])
                return carry

            lax.fori_loop(0, (nch + 1) // 2, pair, 0)

    return gk(table, idx)


def _knn(q_xyz, base_xyz, k):
    B, M, _ = q_xyz.shape
    N = base_xyz.shape[1]
    BM = min(M, 256)
    base_t = jnp.transpose(base_xyz, (0, 2, 1))

    def body(q_ref, bt_ref, o_ref):
        b = pl.program_id(0)
        q = q_ref[0]
        bt = bt_ref[0]
        q0, q1, q2 = q[:, 0:1], q[:, 1:2], q[:, 2:3]
        b0, b1, b2 = bt[0:1, :], bt[1:2, :], bt[2:3, :]
        sq = (q0 * q0 + q1 * q1) + q2 * q2
        sb = (b0 * b0 + b1 * b1) + b2 * b2
        r = lambda t: t.astype(BF16).astype(F32)
        dot = (r(q0) * r(b0) + r(q1) * r(b1)) + r(q2) * r(b2)
        d = (sq + sb) - 2.0 * dot
        iota = lax.broadcasted_iota(jnp.int32, (BM, N), 1)
        kio = lax.broadcasted_iota(jnp.int32, (BM, k), 1)
        acc = jnp.zeros((BM, k), jnp.int32)
        for j in range(k):
            m = jnp.min(d, axis=1, keepdims=True)
            am = jnp.min(jnp.where(d == m, iota, N), axis=1, keepdims=True)
            acc = jnp.where(kio == j, am, acc)
            d = jnp.where(iota == am, jnp.inf, d)
        o_ref[0] = acc + b * N

    return pl.pallas_call(
        body,
        grid=(B, M // BM),
        in_specs=[
            pl.BlockSpec((1, BM, 3), lambda b, i: (b, i, 0)),
            pl.BlockSpec((1, 3, N), lambda b, i: (b, 0, 0)),
        ],
        out_specs=pl.BlockSpec((1, BM, k), lambda b, i: (b, i, 0)),
        out_shape=jax.ShapeDtypeStruct((B, M, k), jnp.int32),
    )(q_xyz, base_t)


def _fps(xyz, npoint):
    B, N, _ = xyz.shape
    LN = min(N, 128)
    NS = N // LN
    xyz_r = jnp.transpose(xyz, (0, 2, 1)).reshape(B, 3, NS, LN)

    def body(xr_ref, o_ref):
        iota2 = (lax.broadcasted_iota(jnp.int32, (B, NS, LN), 1) * LN
                 + lax.broadcasted_iota(jnp.int32, (B, NS, LN), 2))
        iota_p = lax.broadcasted_iota(jnp.int32, (B, npoint), 1)
        xr4 = xr_ref[...]
        neg = jnp.float32(-jnp.inf)

        def step(i, carry):
            far, dist, cent = carry
            cent = jnp.where(iota_p == i, far.reshape(B, 1), cent)
            sel = (iota2 == far)[:, None]
            cm = jnp.max(jnp.max(jnp.where(sel, xr4, neg), axis=3,
                                 keepdims=True), axis=2, keepdims=True)
            d2 = (xr4 - cm) ** 2
            dd = (d2[:, 0] + d2[:, 1]) + d2[:, 2]
            dist = jnp.minimum(dist, dd)
            m = jnp.max(jnp.max(dist, axis=2, keepdims=True), axis=1,
                        keepdims=True)
            far2 = jnp.min(jnp.min(jnp.where(dist == m, iota2, N), axis=2,
                                   keepdims=True), axis=1, keepdims=True)
            return far2, dist, cent

        init = (jnp.zeros((B, 1, 1), jnp.int32),
                jnp.full((B, NS, LN), 1e10, F32),
                jnp.zeros((B, npoint), jnp.int32))
        _, _, cent = lax.fori_loop(0, npoint, step, init)
        boff = lax.broadcasted_iota(jnp.int32, (B, 1, npoint), 0) * N
        o_ref[...] = cent.reshape(B, 1, npoint) + boff

    return pl.pallas_call(
        body,
        grid=(1,),
        in_specs=[
            pl.BlockSpec((B, 3, NS, LN), lambda i: (0, 0, 0, 0)),
        ],
        out_specs=pl.BlockSpec((B, 1, npoint), lambda i: (0, 0, 0)),
        out_shape=jax.ShapeDtypeStruct((B, 1, npoint), jnp.int32),
    )(xyz_r)


def _embed(x, p1, p2):
    B, N, Ci = x.shape
    C1 = p1['w'].shape[0]
    C2 = p2['w'].shape[0]

    def body(x_ref, w1_ref, b1_ref, w2_ref, b2_ref, o_ref):
        h = jnp.maximum(_mm(x_ref[0], w1_ref[...]) + b1_ref[...], 0.0)
        o_ref[0] = _mm(h, w2_ref[...]) + b2_ref[...]

    return pl.pallas_call(
        body,
        grid=(B,),
        in_specs=[
            pl.BlockSpec((1, N, Ci), lambda b: (b, 0, 0)),
            pl.BlockSpec((Ci, C1), lambda b: (0, 0)),
            pl.BlockSpec((1, C1), lambda b: (0, 0)),
            pl.BlockSpec((C1, C2), lambda b: (0, 0)),
            pl.BlockSpec((1, C2), lambda b: (0, 0)),
        ],
        out_specs=pl.BlockSpec((1, N, C2), lambda b: (b, 0, 0)),
        out_shape=jax.ShapeDtypeStruct((B, N, C2), F32),
    )(x, p1['w'].T.astype(BF16), p1['b'].reshape(1, -1),
      p2['w'].T.astype(BF16), p2['b'].reshape(1, -1))


_DM = 512
_TABW = 1152


def _pt_pre(p, feats, xyz):
    B, N, C = feats.shape
    BM = min(N, 256)

    def body(f_ref, xyz_ref, w1_ref, b1_ref, wphi_ref, wpsi_ref, wal_ref,
             q_ref, tab_ref):
        xi = _mm(f_ref[0], w1_ref[...]) + b1_ref[...]
        q_ref[0] = _mm(xi, wphi_ref[...])
        tab_ref[:, 0:_DM] = _mm(xi, wpsi_ref[...])
        tab_ref[:, _DM:2 * _DM] = _mm(xi, wal_ref[...])
        xyzp = jnp.concatenate(
            [xyz_ref[0], jnp.zeros((BM, 125), F32)], axis=1)
        tab_ref[:, 2 * _DM:_TABW] = xyzp

    q, tab = pl.pallas_call(
        body,
        grid=(B, N // BM),
        in_specs=[
            pl.BlockSpec((1, BM, C), lambda b, i: (b, i, 0)),
            pl.BlockSpec((1, BM, 3), lambda b, i: (b, i, 0)),
            pl.BlockSpec((C, _DM), lambda b, i: (0, 0)),
            pl.BlockSpec((1, _DM), lambda b, i: (0, 0)),
            pl.BlockSpec((_DM, _DM), lambda b, i: (0, 0)),
            pl.BlockSpec((_DM, _DM), lambda b, i: (0, 0)),
            pl.BlockSpec((_DM, _DM), lambda b, i: (0, 0)),
        ],
        out_specs=[
            pl.BlockSpec((1, BM, _DM), lambda b, i: (b, i, 0)),
            pl.BlockSpec((BM, _TABW), lambda b, i: (b * (N // BM) + i, 0)),
        ],
        out_shape=[
            jax.ShapeDtypeStruct((B, N, _DM), F32),
            jax.ShapeDtypeStruct((B * N, _TABW), F32),
        ],
    )(feats, xyz,
      p['fc1']['w'].T.astype(BF16), p['fc1']['b'].reshape(1, -1),
      p['phi']['w'].T.astype(BF16), p['psi']['w'].T.astype(BF16),
      p['alpha']['w'].T.astype(BF16))
    return q, tab


def _pt_attn(p, q, xyz, feats, g, k):
    B, N, C = feats.shape
    BM = min(N, 64)
    sqrt_d = float(np.sqrt(_DM))

    def body(q_ref, xyz_ref, f_ref, g_ref, wd1_ref, bd1_ref, wd2_ref, bd2_ref,
             wg1_ref, bg1_ref, wg2_ref, bg2_ref, wf2_ref, bf2_ref, o_ref):
        g2 = g_ref[...]
        kk = g2[:, 0:_DM]
        v = g2[:, _DM:2 * _DM]
        xj = g2[:, 2 * _DM:2 * _DM + 3]
        rel = (xyz_ref[0].reshape(BM, 1, 3) - xj.reshape(BM, k, 3)
               ).reshape(BM * k, 3)
        p1 = jnp.maximum(_mm(rel, wd1_ref[...]) + bd1_ref[...], 0.0)
        pos = _mm(p1, wd2_ref[...]) + bd2_ref[...]
        u = (q_ref[0].reshape(BM, 1, _DM) - kk.reshape(BM, k, _DM)
             ) + pos.reshape(BM, k, _DM)
        a1 = jnp.maximum(
            _mm(u.reshape(BM * k, _DM), wg1_ref[...]) + bg1_ref[...], 0.0)
        e = _mm(a1, wg2_ref[...]) + bg2_ref[...]
        e3 = (e / sqrt_d).reshape(BM, k, _DM)
        m = jnp.max(e3, axis=1, keepdims=True)
        ex = jnp.exp(e3 - m)
        s = jnp.sum(ex, axis=1, keepdims=True)
        attn = ex / s
        res = jnp.sum(attn * (v.reshape(BM, k, _DM) + pos.reshape(BM, k, _DM)),
                      axis=1)
        o_ref[0] = _mm(res, wf2_ref[...]) + bf2_ref[...] + f_ref[0]

    return pl.pallas_call(
        body,
        grid=(B, N // BM),
        in_specs=[
            pl.BlockSpec((1, BM, _DM), lambda b, i: (b, i, 0)),
            pl.BlockSpec((1, BM, 3), lambda b, i: (b, i, 0)),
            pl.BlockSpec((1, BM, C), lambda b, i: (b, i, 0)),
            pl.BlockSpec((BM * k, _TABW), lambda b, i: (b * (N // BM) + i, 0)),
            pl.BlockSpec((3, _DM), lambda b, i: (0, 0)),
            pl.BlockSpec((1, _DM), lambda b, i: (0, 0)),
            pl.BlockSpec((_DM, _DM), lambda b, i: (0, 0)),
            pl.BlockSpec((1, _DM), lambda b, i: (0, 0)),
            pl.BlockSpec((_DM, _DM), lambda b, i: (0, 0)),
            pl.BlockSpec((1, _DM), lambda b, i: (0, 0)),
            pl.BlockSpec((_DM, _DM), lambda b, i: (0, 0)),
            pl.BlockSpec((1, _DM), lambda b, i: (0, 0)),
            pl.BlockSpec((_DM, C), lambda b, i: (0, 0)),
            pl.BlockSpec((1, C), lambda b, i: (0, 0)),
        ],
        out_specs=pl.BlockSpec((1, BM, C), lambda b, i: (b, i, 0)),
        out_shape=jax.ShapeDtypeStruct((B, N, C), F32),
    )(q, xyz, feats, g,
      p['delta1']['w'].T.astype(BF16), p['delta1']['b'].reshape(1, -1),
      p['delta2']['w'].T.astype(BF16), p['delta2']['b'].reshape(1, -1),
      p['gamma1']['w'].T.astype(BF16), p['gamma1']['b'].reshape(1, -1),
      p['gamma2']['w'].T.astype(BF16), p['gamma2']['b'].reshape(1, -1),
      p['fc2']['w'].T.astype(BF16), p['fc2']['b'].reshape(1, -1))


def _pt_block(p, xyz, feats):
    B, N, C = feats.shape
    k = min(16, N)
    idx = _knn(xyz, xyz, k)
    q, tab = _pt_pre(p, feats, xyz)
    g = _sc_gather(tab, idx.reshape(-1))
    return _pt_attn(p, q, xyz, feats, g, k)


def _td_mlp(p, g1, new_xyz2, B, npoint, ns, C, gw):
    l1, l2 = p['layers']
    ch = l1['w'].shape[0]
    chp = max(ch, 128)
    pad = ((0, 0), (0, chp - ch))
    R = B * npoint
    R16 = R * ns
    w1x = jnp.pad(l1['w'][:, :3].T, pad)
    w1p = jnp.pad(l1['w'][:, 3:].T, pad)

    def body(g_ref, nx_ref, wx_ref, wp_ref, b1_ref, ga1_ref, be1_ref,
             w2_ref, b2_ref, ga2_ref, be2_ref, o_ref):
        g2 = g_ref[...]
        pts = g2[:, 0:C]
        xyz_j = g2[:, C:C + 3]
        rel = (xyz_j.reshape(R, ns, 3) - nx_ref[...].reshape(R, 1, 3)
               ).reshape(R16, 3)
        xpre = (_mm(pts, wp_ref[...]) + _mm(rel, wx_ref[...])) + b1_ref[...]
        m1 = jnp.mean(xpre, axis=0, keepdims=True)
        v1 = jnp.mean((xpre - m1) ** 2, axis=0, keepdims=True)
        h = jnp.maximum(
            ga1_ref[...] * (xpre - m1) / jnp.sqrt(v1 + 1e-5) + be1_ref[...], 0.0)
        y = _mm(h, w2_ref[...]) + b2_ref[...]
        m2 = jnp.mean(y, axis=0, keepdims=True)
        v2 = jnp.mean((y - m2) ** 2, axis=0, keepdims=True)
        z = jnp.maximum(
            ga2_ref[...] * (y - m2) / jnp.sqrt(v2 + 1e-5) + be2_ref[...], 0.0)
        o_ref[...] = jnp.max(z.reshape(R, ns, chp), axis=1)

    return pl.pallas_call(
        body,
        grid=(1,),
        in_specs=[
            pl.BlockSpec((R16, gw), lambda i: (0, 0)),
            pl.BlockSpec((R, 3), lambda i: (0, 0)),
            pl.BlockSpec((3, chp), lambda i: (0, 0)),
            pl.BlockSpec((C, chp), lambda i: (0, 0)),
            pl.BlockSpec((1, chp), lambda i: (0, 0)),
            pl.BlockSpec((1, chp), lambda i: (0, 0)),
            pl.BlockSpec((1, chp), lambda i: (0, 0)),
            pl.BlockSpec((chp, chp), lambda i: (0, 0)),
            pl.BlockSpec((1, chp), lambda i: (0, 0)),
            pl.BlockSpec((1, chp), lambda i: (0, 0)),
            pl.BlockSpec((1, chp), lambda i: (0, 0)),
        ],
        out_specs=pl.BlockSpec((R, chp), lambda i: (0, 0)),
        out_shape=jax.ShapeDtypeStruct((R, chp), F32),
    )(g1, new_xyz2,
      w1x.astype(BF16), w1p.astype(BF16),
      jnp.pad(l1['b'].reshape(1, -1), pad),
      jnp.pad(l1['gamma'].reshape(1, -1), pad),
      jnp.pad(l1['beta'].reshape(1, -1), pad),
      jnp.pad(l2['w'].T, ((0, chp - ch), (0, chp - ch))).astype(BF16),
      jnp.pad(l2['b'].reshape(1, -1), pad),
      jnp.pad(l2['gamma'].reshape(1, -1), pad),
      jnp.pad(l2['beta'].reshape(1, -1), pad))


def kernel(input_point_cloud, params):
    x = input_point_cloud
    B, N, _ = x.shape
    xyz = x[..., :3]
    h = _embed(x, params['fc1a'], params['fc1b'])

    points = None
    feats = []
    for i in range(5):
        feat_in = h if i == 0 else points
        p = params['t1'] if i == 0 else params['trs'][i - 1]
        C = feat_in.shape[2]
        Np = feat_in.shape[1]
        k = min(16, Np)
        idx = _knn(xyz, xyz, k)
        q, tab = _pt_pre(p, feat_in, xyz)
        g = _sc_gather(tab, idx.reshape(-1))

        if i < 4:
            npoint = N // 4 ** (i + 1)
            fidx = _fps(xyz, npoint)
            xyz_pad = jnp.pad(xyz.reshape(B * Np, 3), ((0, 0), (0, 125)))
            nxyz_p = _sc_gather(xyz_pad, fidx.reshape(-1))
            new_xyz = nxyz_p[:, :3].reshape(B, npoint, 3)
            ns = min(16, Np)
            idx_td = _knn(new_xyz, xyz, ns)

        points = _pt_attn(p, q, xyz, feat_in, g, k)
        feats.append((xyz, points))

        if i < 4:
            td = params['tds'][i]
            ch = td['layers'][0]['w'].shape[0]
            gw = (C + 3 + 127) // 128 * 128
            tab_td = jnp.pad(
                jnp.concatenate(
                    [points.reshape(B * Np, C), xyz.reshape(B * Np, 3)],
                    axis=1),
                ((0, 0), (0, gw - C - 3)))
            g1 = _sc_gather(tab_td, idx_td.reshape(-1))
            newp = _td_mlp(td, g1, new_xyz.reshape(B * npoint, 3),
                           B, npoint, ns, C, gw)
            xyz = new_xyz
            points = newp[:, :ch].reshape(B, npoint, ch)

    return points, feats


# speedup vs baseline: 1.3143x; 1.0971x over previous
import functools

import numpy as np
import jax
import jax.numpy as jnp
from jax import lax
from jax.experimental import pallas as pl
from jax.experimental.pallas import tpu as pltpu
from jax.experimental.pallas import tpu_sc as plsc

F32 = jnp.float32
BF16 = jnp.bfloat16


def _mm(x, w):
    return jnp.dot(x.astype(BF16), w, preferred_element_type=F32)


def _sc_gather(table, idx):
    R, D = table.shape
    (Bt,) = idx.shape
    W = min(32, Bt // 8)
    bpw = Bt // W
    c = bpw
    while c * D * 4 > 196608 or c > 128:
        c //= 2
    nch = bpw // c
    mesh = plsc.VectorSubcoreMesh(core_axis_name="c", subcore_axis_name="s")

    @functools.partial(
        pl.kernel,
        mesh=mesh,
        out_type=jax.ShapeDtypeStruct((Bt, D), F32),
        scratch_types=[
            pltpu.VMEM((2, c), jnp.int32),
            pltpu.VMEM((2, c, D), F32),
            pltpu.SemaphoreType.DMA,
            pltpu.SemaphoreType.DMA,
        ],
    )
    def gk(table_hbm, idx_hbm, out_hbm, idx_v, rows_v, sem0, sem1):
        wid = lax.axis_index("s") * 2 + lax.axis_index("c")
        sems = (sem0, sem1)

        @pl.when(wid < W)
        def _():
            base0 = wid * bpw
            pltpu.sync_copy(idx_hbm.at[pl.ds(base0, c)], idx_v.at[0])
            cp0 = pltpu.async_copy(table_hbm.at[idx_v.at[0]], rows_v.at[0], sem0)

            def pair(t, carry):
                i0 = 2 * t
                pltpu.make_async_copy(table_hbm.at[idx_v.at[0]], rows_v.at[0],
                                      sem0).wait()

                @pl.when(i0 + 1 < nch)
                def _():
                    pltpu.sync_copy(
                        idx_hbm.at[pl.ds(base0 + (i0 + 1) * c, c)], idx_v.at[1])
                    pltpu.async_copy(table_hbm.at[idx_v.at[1]], rows_v.at[1], sem1)

                pltpu.sync_copy(rows_v.at[0], out_hbm.at[pl.ds(base0 + i0 * c, c)])

                @pl.when(i0 + 1 < nch)
                def _():
                    pltpu.make_async_copy(table_hbm.at[idx_v.at[1]], rows_v.at[1],
                                          sem1).wait()

                    @pl.when(i0 + 2 < nch)
                    def _():
                        pltpu.sync_copy(
                            idx_hbm.at[pl.ds(base0 + (i0 + 2) * c, c)], idx_v.at[0])
                        pltpu.async_copy(table_hbm.at[idx_v.at[0]], rows_v.at[0],
                                         sem0)

                    pltpu.sync_copy(rows_v.at[1],
                                    out_hbm.at[pl.ds(base0 + (i0 + 1) * c, c)])
                return carry

            lax.fori_loop(0, (nch + 1) // 2, pair, 0)

    return gk(table, idx)


def _knn(q_xyz, base_xyz, k):
    B, M, _ = q_xyz.shape
    N = base_xyz.shape[1]
    BM = min(M, 256)
    base_t = jnp.transpose(base_xyz, (0, 2, 1))

    def body(q_ref, bt_ref, o_ref):
        b = pl.program_id(0)
        q = q_ref[0]
        bt = bt_ref[0]
        q0, q1, q2 = q[:, 0:1], q[:, 1:2], q[:, 2:3]
        b0, b1, b2 = bt[0:1, :], bt[1:2, :], bt[2:3, :]
        sq = (q0 * q0 + q1 * q1) + q2 * q2
        sb = (b0 * b0 + b1 * b1) + b2 * b2
        r = lambda t: t.astype(BF16).astype(F32)
        dot = (r(q0) * r(b0) + r(q1) * r(b1)) + r(q2) * r(b2)
        d = (sq + sb) - 2.0 * dot
        iota = lax.broadcasted_iota(jnp.int32, (BM, N), 1)
        kio = lax.broadcasted_iota(jnp.int32, (BM, k), 1)
        acc = jnp.zeros((BM, k), jnp.int32)
        for j in range(k):
            m = jnp.min(d, axis=1, keepdims=True)
            am = jnp.min(jnp.where(d == m, iota, N), axis=1, keepdims=True)
            acc = jnp.where(kio == j, am, acc)
            d = jnp.where(iota == am, jnp.inf, d)
        o_ref[0] = acc + b * N

    return pl.pallas_call(
        body,
        grid=(B, M // BM),
        in_specs=[
            pl.BlockSpec((1, BM, 3), lambda b, i: (b, i, 0)),
            pl.BlockSpec((1, 3, N), lambda b, i: (b, 0, 0)),
        ],
        out_specs=pl.BlockSpec((1, BM, k), lambda b, i: (b, i, 0)),
        out_shape=jax.ShapeDtypeStruct((B, M, k), jnp.int32),
    )(q_xyz, base_t)


def _fps(xyz, npoint):
    B, N, _ = xyz.shape
    LN = min(N, 128)
    NS = N // LN
    xyz_r = jnp.transpose(xyz, (0, 2, 1)).reshape(B, 3, NS, LN)

    def body(xr_ref, o_ref, nx_ref):
        iota2 = (lax.broadcasted_iota(jnp.int32, (B, NS, LN), 1) * LN
                 + lax.broadcasted_iota(jnp.int32, (B, NS, LN), 2))
        iota_p = lax.broadcasted_iota(jnp.int32, (B, npoint), 1)
        xr4 = xr_ref[...]
        neg = jnp.float32(-jnp.inf)

        iota_p3 = lax.broadcasted_iota(jnp.int32, (B, 3, npoint), 2)

        def step(i, carry):
            far, dist, cent, nx = carry
            cent = jnp.where(iota_p == i, far.reshape(B, 1), cent)
            sel = (iota2 == far)[:, None]
            cm = jnp.max(jnp.max(jnp.where(sel, xr4, neg), axis=3,
                                 keepdims=True), axis=2, keepdims=True)
            nx = jnp.where(iota_p3 == i, cm.reshape(B, 3, 1), nx)
            d2 = (xr4 - cm) ** 2
            dd = (d2[:, 0] + d2[:, 1]) + d2[:, 2]
            dist = jnp.minimum(dist, dd)
            m = jnp.max(jnp.max(dist, axis=2, keepdims=True), axis=1,
                        keepdims=True)
            far2 = jnp.min(jnp.min(jnp.where(dist == m, iota2, N), axis=2,
                                   keepdims=True), axis=1, keepdims=True)
            return far2, dist, cent, nx

        init = (jnp.zeros((B, 1, 1), jnp.int32),
                jnp.full((B, NS, LN), 1e10, F32),
                jnp.zeros((B, npoint), jnp.int32),
                jnp.zeros((B, 3, npoint), F32))
        _, _, cent, nx = lax.fori_loop(0, npoint, step, init)
        boff = lax.broadcasted_iota(jnp.int32, (B, 1, npoint), 0) * N
        o_ref[...] = cent.reshape(B, 1, npoint) + boff
        nx_ref[...] = nx

    return pl.pallas_call(
        body,
        grid=(1,),
        in_specs=[
            pl.BlockSpec((B, 3, NS, LN), lambda i: (0, 0, 0, 0)),
        ],
        out_specs=[
            pl.BlockSpec((B, 1, npoint), lambda i: (0, 0, 0)),
            pl.BlockSpec((B, 3, npoint), lambda i: (0, 0, 0)),
        ],
        out_shape=[
            jax.ShapeDtypeStruct((B, 1, npoint), jnp.int32),
            jax.ShapeDtypeStruct((B, 3, npoint), F32),
        ],
    )(xyz_r)


def _embed(x, p1, p2):
    B, N, Ci = x.shape
    C1 = p1['w'].shape[0]
    C2 = p2['w'].shape[0]

    def body(x_ref, w1_ref, b1_ref, w2_ref, b2_ref, o_ref):
        h = jnp.maximum(_mm(x_ref[0], w1_ref[...]) + b1_ref[...], 0.0)
        o_ref[0] = _mm(h, w2_ref[...]) + b2_ref[...]

    return pl.pallas_call(
        body,
        grid=(B,),
        in_specs=[
            pl.BlockSpec((1, N, Ci), lambda b: (b, 0, 0)),
            pl.BlockSpec((Ci, C1), lambda b: (0, 0)),
            pl.BlockSpec((1, C1), lambda b: (0, 0)),
            pl.BlockSpec((C1, C2), lambda b: (0, 0)),
            pl.BlockSpec((1, C2), lambda b: (0, 0)),
        ],
        out_specs=pl.BlockSpec((1, N, C2), lambda b: (b, 0, 0)),
        out_shape=jax.ShapeDtypeStruct((B, N, C2), F32),
    )(x, p1['w'].T.astype(BF16), p1['b'].reshape(1, -1),
      p2['w'].T.astype(BF16), p2['b'].reshape(1, -1))


_DM = 512
_TABW = 1152


def _pt_pre(p, feats, xyz):
    B, N, C = feats.shape
    BM = min(N, 256)

    def body(f_ref, xyz_ref, w1_ref, b1_ref, wphi_ref, wpsi_ref, wal_ref,
             q_ref, tab_ref):
        xi = _mm(f_ref[0], w1_ref[...]) + b1_ref[...]
        q_ref[0] = _mm(xi, wphi_ref[...])
        tab_ref[:, 0:_DM] = _mm(xi, wpsi_ref[...])
        tab_ref[:, _DM:2 * _DM] = _mm(xi, wal_ref[...])
        xyzp = jnp.concatenate(
            [xyz_ref[0], jnp.zeros((BM, 125), F32)], axis=1)
        tab_ref[:, 2 * _DM:_TABW] = xyzp

    q, tab = pl.pallas_call(
        body,
        grid=(B, N // BM),
        in_specs=[
            pl.BlockSpec((1, BM, C), lambda b, i: (b, i, 0)),
            pl.BlockSpec((1, BM, 3), lambda b, i: (b, i, 0)),
            pl.BlockSpec((C, _DM), lambda b, i: (0, 0)),
            pl.BlockSpec((1, _DM), lambda b, i: (0, 0)),
            pl.BlockSpec((_DM, _DM), lambda b, i: (0, 0)),
            pl.BlockSpec((_DM, _DM), lambda b, i: (0, 0)),
            pl.BlockSpec((_DM, _DM), lambda b, i: (0, 0)),
        ],
        out_specs=[
            pl.BlockSpec((1, BM, _DM), lambda b, i: (b, i, 0)),
            pl.BlockSpec((BM, _TABW), lambda b, i: (b * (N // BM) + i, 0)),
        ],
        out_shape=[
            jax.ShapeDtypeStruct((B, N, _DM), F32),
            jax.ShapeDtypeStruct((B * N, _TABW), F32),
        ],
    )(feats, xyz,
      p['fc1']['w'].T.astype(BF16), p['fc1']['b'].reshape(1, -1),
      p['phi']['w'].T.astype(BF16), p['psi']['w'].T.astype(BF16),
      p['alpha']['w'].T.astype(BF16))
    return q, tab


def _pt_attn(p, q, xyz, feats, g, k):
    B, N, C = feats.shape
    BM = min(N, 128)
    sqrt_d = float(np.sqrt(_DM))

    def body(q_ref, xyz_ref, f_ref, g_ref, wd1_ref, bd1_ref, wd2_ref, bd2_ref,
             wg1_ref, bg1_ref, wg2_ref, bg2_ref, wf2_ref, bf2_ref, o_ref):
        g2 = g_ref[...]
        kk = g2[:, 0:_DM]
        v = g2[:, _DM:2 * _DM]
        xj = g2[:, 2 * _DM:2 * _DM + 3]
        rel = (xyz_ref[0].reshape(BM, 1, 3) - xj.reshape(BM, k, 3)
               ).reshape(BM * k, 3)
        p1 = jnp.maximum(_mm(rel, wd1_ref[...]) + bd1_ref[...], 0.0)
        pos = _mm(p1, wd2_ref[...]) + bd2_ref[...]
        u = (q_ref[0].reshape(BM, 1, _DM) - kk.reshape(BM, k, _DM)
             ) + pos.reshape(BM, k, _DM)
        a1 = jnp.maximum(
            _mm(u.reshape(BM * k, _DM), wg1_ref[...]) + bg1_ref[...], 0.0)
        e = _mm(a1, wg2_ref[...]) + bg2_ref[...]
        e3 = (e / sqrt_d).reshape(BM, k, _DM)
        m = jnp.max(e3, axis=1, keepdims=True)
        ex = jnp.exp(e3 - m)
        s = jnp.sum(ex, axis=1, keepdims=True)
        attn = ex / s
        res = jnp.sum(attn * (v.reshape(BM, k, _DM) + pos.reshape(BM, k, _DM)),
                      axis=1)
        o_ref[0] = _mm(res, wf2_ref[...]) + bf2_ref[...] + f_ref[0]

    return pl.pallas_call(
        body,
        grid=(B, N // BM),
        in_specs=[
            pl.BlockSpec((1, BM, _DM), lambda b, i: (b, i, 0)),
            pl.BlockSpec((1, BM, 3), lambda b, i: (b, i, 0)),
            pl.BlockSpec((1, BM, C), lambda b, i: (b, i, 0)),
            pl.BlockSpec((BM * k, _TABW), lambda b, i: (b * (N // BM) + i, 0)),
            pl.BlockSpec((3, _DM), lambda b, i: (0, 0)),
            pl.BlockSpec((1, _DM), lambda b, i: (0, 0)),
            pl.BlockSpec((_DM, _DM), lambda b, i: (0, 0)),
            pl.BlockSpec((1, _DM), lambda b, i: (0, 0)),
            pl.BlockSpec((_DM, _DM), lambda b, i: (0, 0)),
            pl.BlockSpec((1, _DM), lambda b, i: (0, 0)),
            pl.BlockSpec((_DM, _DM), lambda b, i: (0, 0)),
            pl.BlockSpec((1, _DM), lambda b, i: (0, 0)),
            pl.BlockSpec((_DM, C), lambda b, i: (0, 0)),
            pl.BlockSpec((1, C), lambda b, i: (0, 0)),
        ],
        out_specs=pl.BlockSpec((1, BM, C), lambda b, i: (b, i, 0)),
        out_shape=jax.ShapeDtypeStruct((B, N, C), F32),
    )(q, xyz, feats, g,
      p['delta1']['w'].T.astype(BF16), p['delta1']['b'].reshape(1, -1),
      p['delta2']['w'].T.astype(BF16), p['delta2']['b'].reshape(1, -1),
      p['gamma1']['w'].T.astype(BF16), p['gamma1']['b'].reshape(1, -1),
      p['gamma2']['w'].T.astype(BF16), p['gamma2']['b'].reshape(1, -1),
      p['fc2']['w'].T.astype(BF16), p['fc2']['b'].reshape(1, -1))


def _pt_block(p, xyz, feats):
    B, N, C = feats.shape
    k = min(16, N)
    idx = _knn(xyz, xyz, k)
    q, tab = _pt_pre(p, feats, xyz)
    g = _sc_gather(tab, idx.reshape(-1))
    return _pt_attn(p, q, xyz, feats, g, k)


def _td_mlp(p, g1, new_xyz2, B, npoint, ns, C, gw):
    l1, l2 = p['layers']
    ch = l1['w'].shape[0]
    chp = max(ch, 128)
    pad = ((0, 0), (0, chp - ch))
    R = B * npoint
    R16 = R * ns
    w1x = jnp.pad(l1['w'][:, :3].T, pad)
    w1p = jnp.pad(l1['w'][:, 3:].T, pad)

    def body(g_ref, nx_ref, wx_ref, wp_ref, b1_ref, ga1_ref, be1_ref,
             w2_ref, b2_ref, ga2_ref, be2_ref, o_ref):
        g2 = g_ref[...]
        pts = g2[:, 0:C]
        xyz_j = g2[:, C:C + 3]
        rel = (xyz_j.reshape(R, ns, 3) - nx_ref[...].reshape(R, 1, 3)
               ).reshape(R16, 3)
        xpre = (_mm(pts, wp_ref[...]) + _mm(rel, wx_ref[...])) + b1_ref[...]
        m1 = jnp.mean(xpre, axis=0, keepdims=True)
        v1 = jnp.mean((xpre - m1) ** 2, axis=0, keepdims=True)
        h = jnp.maximum(
            ga1_ref[...] * (xpre - m1) / jnp.sqrt(v1 + 1e-5) + be1_ref[...], 0.0)
        y = _mm(h, w2_ref[...]) + b2_ref[...]
        m2 = jnp.mean(y, axis=0, keepdims=True)
        v2 = jnp.mean((y - m2) ** 2, axis=0, keepdims=True)
        z = jnp.maximum(
            ga2_ref[...] * (y - m2) / jnp.sqrt(v2 + 1e-5) + be2_ref[...], 0.0)
        o_ref[...] = jnp.max(z.reshape(R, ns, chp), axis=1)

    return pl.pallas_call(
        body,
        grid=(1,),
        in_specs=[
            pl.BlockSpec((R16, gw), lambda i: (0, 0)),
            pl.BlockSpec((R, 3), lambda i: (0, 0)),
            pl.BlockSpec((3, chp), lambda i: (0, 0)),
            pl.BlockSpec((C, chp), lambda i: (0, 0)),
            pl.BlockSpec((1, chp), lambda i: (0, 0)),
            pl.BlockSpec((1, chp), lambda i: (0, 0)),
            pl.BlockSpec((1, chp), lambda i: (0, 0)),
            pl.BlockSpec((chp, chp), lambda i: (0, 0)),
            pl.BlockSpec((1, chp), lambda i: (0, 0)),
            pl.BlockSpec((1, chp), lambda i: (0, 0)),
            pl.BlockSpec((1, chp), lambda i: (0, 0)),
        ],
        out_specs=pl.BlockSpec((R, chp), lambda i: (0, 0)),
        out_shape=jax.ShapeDtypeStruct((R, chp), F32),
    )(g1, new_xyz2,
      w1x.astype(BF16), w1p.astype(BF16),
      jnp.pad(l1['b'].reshape(1, -1), pad),
      jnp.pad(l1['gamma'].reshape(1, -1), pad),
      jnp.pad(l1['beta'].reshape(1, -1), pad),
      jnp.pad(l2['w'].T, ((0, chp - ch), (0, chp - ch))).astype(BF16),
      jnp.pad(l2['b'].reshape(1, -1), pad),
      jnp.pad(l2['gamma'].reshape(1, -1), pad),
      jnp.pad(l2['beta'].reshape(1, -1), pad))


def kernel(input_point_cloud, params):
    x = input_point_cloud
    B, N, _ = x.shape
    xyz = x[..., :3]
    h = _embed(x, params['fc1a'], params['fc1b'])

    points = None
    feats = []
    for i in range(5):
        feat_in = h if i == 0 else points
        p = params['t1'] if i == 0 else params['trs'][i - 1]
        C = feat_in.shape[2]
        Np = feat_in.shape[1]
        k = min(16, Np)
        idx = _knn(xyz, xyz, k)
        q, tab = _pt_pre(p, feat_in, xyz)
        g = _sc_gather(tab, idx.reshape(-1))

        if i < 4:
            npoint = N // 4 ** (i + 1)
            fidx, nx3 = _fps(xyz, npoint)
            new_xyz = jnp.transpose(nx3, (0, 2, 1))
            ns = min(16, Np)
            idx_td = _knn(new_xyz, xyz, ns)

        points = _pt_attn(p, q, xyz, feat_in, g, k)
        feats.append((xyz, points))

        if i < 4:
            td = params['tds'][i]
            ch = td['layers'][0]['w'].shape[0]
            gw = (C + 3 + 127) // 128 * 128
            tab_td = jnp.pad(
                jnp.concatenate(
                    [points.reshape(B * Np, C), xyz.reshape(B * Np, 3)],
                    axis=1),
                ((0, 0), (0, gw - C - 3)))
            g1 = _sc_gather(tab_td, idx_td.reshape(-1))
            newp = _td_mlp(td, g1, new_xyz.reshape(B * npoint, 3),
                           B, npoint, ns, C, gw)
            xyz = new_xyz
            points = newp[:, :ch].reshape(B, npoint, ch)

    return points, feats


# speedup vs baseline: 1.4143x; 1.0761x over previous
import functools

import numpy as np
import jax
import jax.numpy as jnp
from jax import lax
from jax.experimental import pallas as pl
from jax.experimental.pallas import tpu as pltpu
from jax.experimental.pallas import tpu_sc as plsc

F32 = jnp.float32
BF16 = jnp.bfloat16


def _mm(x, w):
    return jnp.dot(x.astype(BF16), w, preferred_element_type=F32)


def _sc_gather(table, idx):
    R, D = table.shape
    (Bt,) = idx.shape
    W = min(32, Bt // 8)
    bpw = Bt // W
    c = bpw
    while c * D * 4 > 196608 or c > 128:
        c //= 2
    nch = bpw // c
    mesh = plsc.VectorSubcoreMesh(core_axis_name="c", subcore_axis_name="s")

    @functools.partial(
        pl.kernel,
        mesh=mesh,
        out_type=jax.ShapeDtypeStruct((Bt, D), F32),
        scratch_types=[
            pltpu.VMEM((2, c), jnp.int32),
            pltpu.VMEM((2, c, D), F32),
            pltpu.SemaphoreType.DMA,
            pltpu.SemaphoreType.DMA,
        ],
    )
    def gk(table_hbm, idx_hbm, out_hbm, idx_v, rows_v, sem0, sem1):
        wid = lax.axis_index("s") * 2 + lax.axis_index("c")
        sems = (sem0, sem1)

        @pl.when(wid < W)
        def _():
            base0 = wid * bpw
            pltpu.sync_copy(idx_hbm.at[pl.ds(base0, c)], idx_v.at[0])
            cp0 = pltpu.async_copy(table_hbm.at[idx_v.at[0]], rows_v.at[0], sem0)

            def pair(t, carry):
                i0 = 2 * t
                pltpu.make_async_copy(table_hbm.at[idx_v.at[0]], rows_v.at[0],
                                      sem0).wait()

                @pl.when(i0 + 1 < nch)
                def _():
                    pltpu.sync_copy(
                        idx_hbm.at[pl.ds(base0 + (i0 + 1) * c, c)], idx_v.at[1])
                    pltpu.async_copy(table_hbm.at[idx_v.at[1]], rows_v.at[1], sem1)

                pltpu.sync_copy(rows_v.at[0], out_hbm.at[pl.ds(base0 + i0 * c, c)])

                @pl.when(i0 + 1 < nch)
                def _():
                    pltpu.make_async_copy(table_hbm.at[idx_v.at[1]], rows_v.at[1],
                                          sem1).wait()

                    @pl.when(i0 + 2 < nch)
                    def _():
                        pltpu.sync_copy(
                            idx_hbm.at[pl.ds(base0 + (i0 + 2) * c, c)], idx_v.at[0])
                        pltpu.async_copy(table_hbm.at[idx_v.at[0]], rows_v.at[0],
                                         sem0)

                    pltpu.sync_copy(rows_v.at[1],
                                    out_hbm.at[pl.ds(base0 + (i0 + 1) * c, c)])
                return carry

            lax.fori_loop(0, (nch + 1) // 2, pair, 0)

    return gk(table, idx)


def _knn(q_xyz, base_xyz, k):
    B, M, _ = q_xyz.shape
    N = base_xyz.shape[1]
    BM = min(M, 256)
    base_t = jnp.transpose(base_xyz, (0, 2, 1))

    def body(q_ref, bt_ref, o_ref):
        b = pl.program_id(0)
        q = q_ref[0]
        bt = bt_ref[0]
        q0, q1, q2 = q[:, 0:1], q[:, 1:2], q[:, 2:3]
        b0, b1, b2 = bt[0:1, :], bt[1:2, :], bt[2:3, :]
        sq = (q0 * q0 + q1 * q1) + q2 * q2
        sb = (b0 * b0 + b1 * b1) + b2 * b2
        r = lambda t: t.astype(BF16).astype(F32)
        dot = (r(q0) * r(b0) + r(q1) * r(b1)) + r(q2) * r(b2)
        d = (sq + sb) - 2.0 * dot
        iota = lax.broadcasted_iota(jnp.int32, (BM, N), 1).astype(F32)
        kio = lax.broadcasted_iota(jnp.int32, (BM, k), 1)
        fn = jnp.float32(N)
        acc = jnp.zeros((BM, k), F32)
        for j in range(k):
            m = jnp.min(d, axis=1, keepdims=True)
            am = jnp.min(jnp.where(d == m, iota, fn), axis=1, keepdims=True)
            acc = jnp.where(kio == j, am, acc)
            d = jnp.where(iota == am, jnp.inf, d)
        o_ref[0] = acc.astype(jnp.int32) + b * N

    return pl.pallas_call(
        body,
        grid=(B, M // BM),
        in_specs=[
            pl.BlockSpec((1, BM, 3), lambda b, i: (b, i, 0)),
            pl.BlockSpec((1, 3, N), lambda b, i: (b, 0, 0)),
        ],
        out_specs=pl.BlockSpec((1, BM, k), lambda b, i: (b, i, 0)),
        out_shape=jax.ShapeDtypeStruct((B, M, k), jnp.int32),
    )(q_xyz, base_t)


def _fps(xyz, npoint):
    B, N, _ = xyz.shape
    LN = min(N, 128)
    NS = N // LN
    xyz_r = jnp.transpose(xyz, (0, 2, 1)).reshape(B, 3, NS, LN)

    def body(xr_ref, o_ref, nx_ref):
        iota2 = (lax.broadcasted_iota(jnp.int32, (B, NS, LN), 1) * LN
                 + lax.broadcasted_iota(jnp.int32, (B, NS, LN), 2)).astype(F32)
        iota_p = lax.broadcasted_iota(jnp.int32, (B, npoint), 1)
        fn = jnp.float32(N)
        xr4 = xr_ref[...]
        neg = jnp.float32(-jnp.inf)

        iota_p3 = lax.broadcasted_iota(jnp.int32, (B, 3, npoint), 2)

        def step(i, carry):
            far, dist, cent, nx = carry
            cent = jnp.where(iota_p == i, far.reshape(B, 1), cent)
            sel = (iota2 == far)[:, None]
            cm = jnp.max(jnp.max(jnp.where(sel, xr4, neg), axis=3,
                                 keepdims=True), axis=2, keepdims=True)
            nx = jnp.where(iota_p3 == i, cm.reshape(B, 3, 1), nx)
            d2 = (xr4 - cm) ** 2
            dd = (d2[:, 0] + d2[:, 1]) + d2[:, 2]
            dist = jnp.minimum(dist, dd)
            m = jnp.max(jnp.max(dist, axis=2, keepdims=True), axis=1,
                        keepdims=True)
            far2 = jnp.min(jnp.min(jnp.where(dist == m, iota2, fn), axis=2,
                                   keepdims=True), axis=1, keepdims=True)
            return far2, dist, cent, nx

        init = (jnp.zeros((B, 1, 1), F32),
                jnp.full((B, NS, LN), 1e10, F32),
                jnp.zeros((B, npoint), F32),
                jnp.zeros((B, 3, npoint), F32))
        _, _, cent, nx = lax.fori_loop(0, npoint, step, init)
        boff = lax.broadcasted_iota(jnp.int32, (B, 1, npoint), 0) * N
        o_ref[...] = cent.astype(jnp.int32).reshape(B, 1, npoint) + boff
        nx_ref[...] = nx

    return pl.pallas_call(
        body,
        grid=(1,),
        in_specs=[
            pl.BlockSpec((B, 3, NS, LN), lambda i: (0, 0, 0, 0)),
        ],
        out_specs=[
            pl.BlockSpec((B, 1, npoint), lambda i: (0, 0, 0)),
            pl.BlockSpec((B, 3, npoint), lambda i: (0, 0, 0)),
        ],
        out_shape=[
            jax.ShapeDtypeStruct((B, 1, npoint), jnp.int32),
            jax.ShapeDtypeStruct((B, 3, npoint), F32),
        ],
    )(xyz_r)


def _embed(x, p1, p2):
    B, N, Ci = x.shape
    C1 = p1['w'].shape[0]
    C2 = p2['w'].shape[0]

    def body(x_ref, w1_ref, b1_ref, w2_ref, b2_ref, o_ref):
        h = jnp.maximum(_mm(x_ref[0], w1_ref[...]) + b1_ref[...], 0.0)
        o_ref[0] = _mm(h, w2_ref[...]) + b2_ref[...]

    return pl.pallas_call(
        body,
        grid=(B,),
        in_specs=[
            pl.BlockSpec((1, N, Ci), lambda b: (b, 0, 0)),
            pl.BlockSpec((Ci, C1), lambda b: (0, 0)),
            pl.BlockSpec((1, C1), lambda b: (0, 0)),
            pl.BlockSpec((C1, C2), lambda b: (0, 0)),
            pl.BlockSpec((1, C2), lambda b: (0, 0)),
        ],
        out_specs=pl.BlockSpec((1, N, C2), lambda b: (b, 0, 0)),
        out_shape=jax.ShapeDtypeStruct((B, N, C2), F32),
    )(x, p1['w'].T.astype(BF16), p1['b'].reshape(1, -1),
      p2['w'].T.astype(BF16), p2['b'].reshape(1, -1))


_DM = 512
_TABW = 1152


def _pt_pre(p, feats, xyz):
    B, N, C = feats.shape
    BM = min(N, 256)

    def body(f_ref, xyz_ref, w1_ref, b1_ref, wphi_ref, wpsi_ref, wal_ref,
             q_ref, tab_ref):
        xi = _mm(f_ref[0], w1_ref[...]) + b1_ref[...]
        q_ref[0] = _mm(xi, wphi_ref[...])
        tab_ref[:, 0:_DM] = _mm(xi, wpsi_ref[...])
        tab_ref[:, _DM:2 * _DM] = _mm(xi, wal_ref[...])
        xyzp = jnp.concatenate(
            [xyz_ref[0], jnp.zeros((BM, 125), F32)], axis=1)
        tab_ref[:, 2 * _DM:_TABW] = xyzp

    q, tab = pl.pallas_call(
        body,
        grid=(B, N // BM),
        in_specs=[
            pl.BlockSpec((1, BM, C), lambda b, i: (b, i, 0)),
            pl.BlockSpec((1, BM, 3), lambda b, i: (b, i, 0)),
            pl.BlockSpec((C, _DM), lambda b, i: (0, 0)),
            pl.BlockSpec((1, _DM), lambda b, i: (0, 0)),
            pl.BlockSpec((_DM, _DM), lambda b, i: (0, 0)),
            pl.BlockSpec((_DM, _DM), lambda b, i: (0, 0)),
            pl.BlockSpec((_DM, _DM), lambda b, i: (0, 0)),
        ],
        out_specs=[
            pl.BlockSpec((1, BM, _DM), lambda b, i: (b, i, 0)),
            pl.BlockSpec((BM, _TABW), lambda b, i: (b * (N // BM) + i, 0)),
        ],
        out_shape=[
            jax.ShapeDtypeStruct((B, N, _DM), F32),
            jax.ShapeDtypeStruct((B * N, _TABW), F32),
        ],
    )(feats, xyz,
      p['fc1']['w'].T.astype(BF16), p['fc1']['b'].reshape(1, -1),
      p['phi']['w'].T.astype(BF16), p['psi']['w'].T.astype(BF16),
      p['alpha']['w'].T.astype(BF16))
    return q, tab


def _pt_attn(p, q, xyz, feats, g, k):
    B, N, C = feats.shape
    BM = min(N, 128)
    sqrt_d = float(np.sqrt(_DM))

    def body(q_ref, xyz_ref, f_ref, g_ref, wd1_ref, bd1_ref, wd2_ref, bd2_ref,
             wg1_ref, bg1_ref, wg2_ref, bg2_ref, wf2_ref, bf2_ref, o_ref):
        g2 = g_ref[...]
        kk = g2[:, 0:_DM]
        v = g2[:, _DM:2 * _DM]
        xj = g2[:, 2 * _DM:2 * _DM + 3]
        rel = (xyz_ref[0].reshape(BM, 1, 3) - xj.reshape(BM, k, 3)
               ).reshape(BM * k, 3)
        p1 = jnp.maximum(_mm(rel, wd1_ref[...]) + bd1_ref[...], 0.0)
        pos = _mm(p1, wd2_ref[...]) + bd2_ref[...]
        u = (q_ref[0].reshape(BM, 1, _DM) - kk.reshape(BM, k, _DM)
             ) + pos.reshape(BM, k, _DM)
        a1 = jnp.maximum(
            _mm(u.reshape(BM * k, _DM), wg1_ref[...]) + bg1_ref[...], 0.0)
        e = _mm(a1, wg2_ref[...]) + bg2_ref[...]
        e3 = (e / sqrt_d).reshape(BM, k, _DM)
        m = jnp.max(e3, axis=1, keepdims=True)
        ex = jnp.exp(e3 - m)
        s = jnp.sum(ex, axis=1, keepdims=True)
        attn = ex / s
        res = jnp.sum(attn * (v.reshape(BM, k, _DM) + pos.reshape(BM, k, _DM)),
                      axis=1)
        o_ref[0] = _mm(res, wf2_ref[...]) + bf2_ref[...] + f_ref[0]

    return pl.pallas_call(
        body,
        grid=(B, N // BM),
        in_specs=[
            pl.BlockSpec((1, BM, _DM), lambda b, i: (b, i, 0)),
            pl.BlockSpec((1, BM, 3), lambda b, i: (b, i, 0)),
            pl.BlockSpec((1, BM, C), lambda b, i: (b, i, 0)),
            pl.BlockSpec((BM * k, _TABW), lambda b, i: (b * (N // BM) + i, 0)),
            pl.BlockSpec((3, _DM), lambda b, i: (0, 0)),
            pl.BlockSpec((1, _DM), lambda b, i: (0, 0)),
            pl.BlockSpec((_DM, _DM), lambda b, i: (0, 0)),
            pl.BlockSpec((1, _DM), lambda b, i: (0, 0)),
            pl.BlockSpec((_DM, _DM), lambda b, i: (0, 0)),
            pl.BlockSpec((1, _DM), lambda b, i: (0, 0)),
            pl.BlockSpec((_DM, _DM), lambda b, i: (0, 0)),
            pl.BlockSpec((1, _DM), lambda b, i: (0, 0)),
            pl.BlockSpec((_DM, C), lambda b, i: (0, 0)),
            pl.BlockSpec((1, C), lambda b, i: (0, 0)),
        ],
        out_specs=pl.BlockSpec((1, BM, C), lambda b, i: (b, i, 0)),
        out_shape=jax.ShapeDtypeStruct((B, N, C), F32),
    )(q, xyz, feats, g,
      p['delta1']['w'].T.astype(BF16), p['delta1']['b'].reshape(1, -1),
      p['delta2']['w'].T.astype(BF16), p['delta2']['b'].reshape(1, -1),
      p['gamma1']['w'].T.astype(BF16), p['gamma1']['b'].reshape(1, -1),
      p['gamma2']['w'].T.astype(BF16), p['gamma2']['b'].reshape(1, -1),
      p['fc2']['w'].T.astype(BF16), p['fc2']['b'].reshape(1, -1))


def _pt_block(p, xyz, feats):
    B, N, C = feats.shape
    k = min(16, N)
    idx = _knn(xyz, xyz, k)
    q, tab = _pt_pre(p, feats, xyz)
    g = _sc_gather(tab, idx.reshape(-1))
    return _pt_attn(p, q, xyz, feats, g, k)


def _td_mlp(p, g1, new_xyz2, B, npoint, ns, C, gw):
    l1, l2 = p['layers']
    ch = l1['w'].shape[0]
    chp = max(ch, 128)
    pad = ((0, 0), (0, chp - ch))
    R = B * npoint
    R16 = R * ns
    w1x = jnp.pad(l1['w'][:, :3].T, pad)
    w1p = jnp.pad(l1['w'][:, 3:].T, pad)

    def body(g_ref, nx_ref, wx_ref, wp_ref, b1_ref, ga1_ref, be1_ref,
             w2_ref, b2_ref, ga2_ref, be2_ref, o_ref):
        g2 = g_ref[...]
        pts = g2[:, 0:C]
        xyz_j = g2[:, C:C + 3]
        rel = (xyz_j.reshape(R, ns, 3) - nx_ref[...].reshape(R, 1, 3)
               ).reshape(R16, 3)
        xpre = (_mm(pts, wp_ref[...]) + _mm(rel, wx_ref[...])) + b1_ref[...]
        m1 = jnp.mean(xpre, axis=0, keepdims=True)
        v1 = jnp.mean((xpre - m1) ** 2, axis=0, keepdims=True)
        h = jnp.maximum(
            ga1_ref[...] * (xpre - m1) / jnp.sqrt(v1 + 1e-5) + be1_ref[...], 0.0)
        y = _mm(h, w2_ref[...]) + b2_ref[...]
        m2 = jnp.mean(y, axis=0, keepdims=True)
        v2 = jnp.mean((y - m2) ** 2, axis=0, keepdims=True)
        z = jnp.maximum(
            ga2_ref[...] * (y - m2) / jnp.sqrt(v2 + 1e-5) + be2_ref[...], 0.0)
        o_ref[...] = jnp.max(z.reshape(R, ns, chp), axis=1)

    return pl.pallas_call(
        body,
        grid=(1,),
        in_specs=[
            pl.BlockSpec((R16, gw), lambda i: (0, 0)),
            pl.BlockSpec((R, 3), lambda i: (0, 0)),
            pl.BlockSpec((3, chp), lambda i: (0, 0)),
            pl.BlockSpec((C, chp), lambda i: (0, 0)),
            pl.BlockSpec((1, chp), lambda i: (0, 0)),
            pl.BlockSpec((1, chp), lambda i: (0, 0)),
            pl.BlockSpec((1, chp), lambda i: (0, 0)),
            pl.BlockSpec((chp, chp), lambda i: (0, 0)),
            pl.BlockSpec((1, chp), lambda i: (0, 0)),
            pl.BlockSpec((1, chp), lambda i: (0, 0)),
            pl.BlockSpec((1, chp), lambda i: (0, 0)),
        ],
        out_specs=pl.BlockSpec((R, chp), lambda i: (0, 0)),
        out_shape=jax.ShapeDtypeStruct((R, chp), F32),
    )(g1, new_xyz2,
      w1x.astype(BF16), w1p.astype(BF16),
      jnp.pad(l1['b'].reshape(1, -1), pad),
      jnp.pad(l1['gamma'].reshape(1, -1), pad),
      jnp.pad(l1['beta'].reshape(1, -1), pad),
      jnp.pad(l2['w'].T, ((0, chp - ch), (0, chp - ch))).astype(BF16),
      jnp.pad(l2['b'].reshape(1, -1), pad),
      jnp.pad(l2['gamma'].reshape(1, -1), pad),
      jnp.pad(l2['beta'].reshape(1, -1), pad))


def kernel(input_point_cloud, params):
    x = input_point_cloud
    B, N, _ = x.shape
    xyz = x[..., :3]
    h = _embed(x, params['fc1a'], params['fc1b'])

    points = None
    feats = []
    for i in range(5):
        feat_in = h if i == 0 else points
        p = params['t1'] if i == 0 else params['trs'][i - 1]
        C = feat_in.shape[2]
        Np = feat_in.shape[1]
        k = min(16, Np)
        idx = _knn(xyz, xyz, k)
        q, tab = _pt_pre(p, feat_in, xyz)
        g = _sc_gather(tab, idx.reshape(-1))

        if i < 4:
            npoint = N // 4 ** (i + 1)
            fidx, nx3 = _fps(xyz, npoint)
            new_xyz = jnp.transpose(nx3, (0, 2, 1))
            ns = min(16, Np)
            idx_td = _knn(new_xyz, xyz, ns)

        points = _pt_attn(p, q, xyz, feat_in, g, k)
        feats.append((xyz, points))

        if i < 4:
            td = params['tds'][i]
            ch = td['layers'][0]['w'].shape[0]
            gw = (C + 3 + 127) // 128 * 128
            tab_td = jnp.pad(
                jnp.concatenate(
                    [points.reshape(B * Np, C), xyz.reshape(B * Np, 3)],
                    axis=1),
                ((0, 0), (0, gw - C - 3)))
            g1 = _sc_gather(tab_td, idx_td.reshape(-1))
            newp = _td_mlp(td, g1, new_xyz.reshape(B * npoint, 3),
                           B, npoint, ns, C, gw)
            xyz = new_xyz
            points = newp[:, :ch].reshape(B, npoint, ch)

    return points, feats
